# baseline (speedup 1.0000x reference)
import functools
import jax
import jax.numpy as jnp
from jax.experimental import pallas as pl
from jax.experimental.pallas import tpu as pltpu

F32 = jnp.float32
NEG = -3e38
CPAD = 50176
CT = 1024
QT = 128


def _ln(x, g, b):
    m = jnp.mean(x, axis=-1, keepdims=True)
    v = jnp.mean((x - m) ** 2, axis=-1, keepdims=True)
    return (x - m) * jax.lax.rsqrt(v + 1e-5) * g + b


def _encode_body(x_ref, wlin, blin, w1a, b1a, w2a, b2a, g1, be1, w1b, b1b,
                 w2b, b2b, gn, bn, wo, bo, xn_out, o_out):
    h = jnp.dot(x_ref[...], wlin[...], preferred_element_type=F32) + blin[...]
    z = jnp.maximum(jnp.dot(h, w1a[...], preferred_element_type=F32) + b1a[...], 0.0)
    h = h + jnp.dot(z, w2a[...], preferred_element_type=F32) + b2a[...]
    zn = _ln(h, g1[...], be1[...])
    z = jnp.maximum(jnp.dot(zn, w1b[...], preferred_element_type=F32) + b1b[...], 0.0)
    h = h + jnp.dot(z, w2b[...], preferred_element_type=F32) + b2b[...]
    hn = _ln(h, gn[...], bn[...])
    xn_out[...] = hn.astype(jnp.bfloat16)
    o_out[...] = jnp.dot(hn, wo[...], preferred_element_type=F32) + bo[...]


def _encode_q_body(x_ref, wlin, blin, w1a, b1a, w2a, b2a, g1, be1, w1b, b1b,
                   w2b, b2b, gn, bn, wq, bq, x_out, q_out):
    h = jnp.dot(x_ref[...], wlin[...], preferred_element_type=F32) + blin[...]
    z = jnp.maximum(jnp.dot(h, w1a[...], preferred_element_type=F32) + b1a[...], 0.0)
    h = h + jnp.dot(z, w2a[...], preferred_element_type=F32) + b2a[...]
    zn = _ln(h, g1[...], be1[...])
    z = jnp.maximum(jnp.dot(zn, w1b[...], preferred_element_type=F32) + b1b[...], 0.0)
    h = h + jnp.dot(z, w2b[...], preferred_element_type=F32) + b2b[...]
    hn = _ln(h, gn[...], bn[...])
    x_out[...] = h
    q_out[...] = jnp.dot(hn, wq[...], preferred_element_type=F32) + bq[...]


def _scores_body(q_ref, ck_ref, out_ref):
    j = pl.program_id(1)
    s = jax.lax.dot_general(q_ref[...], ck_ref[...],
                            (((1,), (1,)), ((), ())),
                            preferred_element_type=F32)
    col = j * CT + jax.lax.broadcasted_iota(jnp.int32, s.shape, 1)
    out_ref[...] = jnp.where(col >= 50000, NEG, s)


def _inv_sortable(u):
    b = jnp.where(u >= 0, u, u ^ jnp.int32(0x7FFFFFFF))
    return jax.lax.bitcast_convert_type(b, F32)


def _thresh_body(s_ref, t_out, m_out, k):
    s = s_ref[...]
    rows, n = s.shape
    rowmax = jnp.max(s, axis=1, keepdims=True)

    def sortable(x):
        bx = jax.lax.bitcast_convert_type(x, jnp.int32)
        return jnp.where(bx >= 0, bx, bx ^ jnp.int32(0x7FFFFFFF))

    cmax = jnp.max(s.reshape(rows, n // 512, 512), axis=1)

    def body_sm(it, t):
        cand = t + jax.lax.shift_left(jnp.int32(1), 30 - it)
        cnt = jnp.sum((cmax >= _inv_sortable(cand)).astype(F32), axis=1,
                      keepdims=True)
        return jnp.where(cnt >= k, cand, t)

    def body(it, t):
        cand = t + jax.lax.shift_left(jnp.int32(1), pmax - it)
        cnt = jnp.sum((s >= _inv_sortable(cand)).astype(F32), axis=1,
                      keepdims=True)
        return jnp.where(cnt >= k, cand, t)

    cnt0 = jnp.sum((s >= 0.0).astype(F32), axis=1, keepdims=True)
    t0 = jnp.where(cnt0 >= k, jnp.int32(0), jnp.int32(-2147483647 - 1))
    u_lo = jax.lax.fori_loop(0, 31, body_sm, t0)
    u_hi = sortable(rowmax)
    u_hi = jnp.where(t0 < 0, jnp.minimum(u_hi, -1), u_hi)
    u_hi = jnp.maximum(u_hi, u_lo)
    gap = (u_lo ^ u_hi).astype(F32)
    e = (jax.lax.bitcast_convert_type(jnp.max(gap), jnp.int32) >> 23) - 127
    pmax = jnp.clip(e + 1, 0, 30)
    mask_hi = ~(jax.lax.shift_left(jnp.int32(2), pmax) - 1)
    t = jax.lax.fori_loop(0, pmax + 1, body, u_lo & mask_hi)
    t_out[...] = _inv_sortable(t)
    m_out[...] = rowmax


def _wsum_body(s_ref, xn_ref, y_ref, t_ref, m_ref, s_out, wy_out, d_out,
               s_acc, wy_acc, d_acc, nj):
    j = pl.program_id(1)

    @pl.when(j == 0)
    def _():
        s_acc[...] = jnp.zeros_like(s_acc)
        wy_acc[...] = jnp.zeros_like(wy_acc)
        d_acc[...] = jnp.zeros_like(d_acc)

    s = s_ref[...]
    w = jnp.where(s >= t_ref[...],
                  jnp.exp((s - m_ref[...]) * (1.0 / 16.0)), 0.0)
    s_acc[...] += jnp.dot(w.astype(jnp.bfloat16), xn_ref[...],
                          preferred_element_type=F32)
    wy_acc[...] += jnp.dot(w, y_ref[...], preferred_element_type=F32)
    d_acc[...] += jnp.sum(w, axis=1, keepdims=True)

    @pl.when(j == nj - 1)
    def _():
        s_out[...] = s_acc[...]
        wy_out[...] = wy_acc[...]
        d_out[...] = d_acc[...]


def _head_body(x_ref, s_ref, wy_ref, d_ref, wv, bv, wl, bl,
               g1, be1, w11, b11, w21, b21, g2, be2, w12, b12, w22, b22,
               hg, hb, wh, bh, out_ref):
    dinv = 1.0 / d_ref[...]
    sn = s_ref[...] * dinv
    ctx = (jnp.dot(sn, wv[...], preferred_element_type=F32) + bv[...]
           + (wy_ref[...] * dinv) * wl[...] + bl[...])
    h = x_ref[...] + ctx
    z = _ln(h, g1[...], be1[...])
    z = jnp.maximum(jnp.dot(z, w11[...], preferred_element_type=F32) + b11[...], 0.0)
    h = h + jnp.dot(z, w21[...], preferred_element_type=F32) + b21[...]
    z = _ln(h, g2[...], be2[...])
    z = jnp.maximum(jnp.dot(z, w12[...], preferred_element_type=F32) + b12[...], 0.0)
    h = h + jnp.dot(z, w22[...], preferred_element_type=F32) + b22[...]
    hn = jnp.maximum(_ln(h, hg[...], hb[...]), 0.0)
    out_ref[...] = jnp.dot(hn, wh[...], preferred_element_type=F32) + bh[...]


def _full(shape):
    n = len(shape)
    return pl.BlockSpec(shape, lambda *a: (0,) * n)


def kernel(x_num, candidate_x_num, candidate_y, params, context_size):
    B, D_IN = x_num.shape
    NC = candidate_x_num.shape[0]
    D = params['norm'][0].shape[0]

    r2 = lambda v: v.reshape(1, -1)
    wlin, blin = params['lin'][0].T, r2(params['lin'][1])
    e0, e1 = params['enc']
    w1a, b1a = e0['l1'][0].T, r2(e0['l1'][1])
    w2a, b2a = e0['l2'][0].T, r2(e0['l2'][1])
    g1, be1 = r2(e1['ln'][0]), r2(e1['ln'][1])
    w1b, b1b = e1['l1'][0].T, r2(e1['l1'][1])
    w2b, b2b = e1['l2'][0].T, r2(e1['l2'][1])
    gn, bn = r2(params['norm'][0]), r2(params['norm'][1])
    wk, bk = params['K'][0].T, r2(params['K'][1])
    wq, bq = params['Q'][0].T, r2(params['Q'][1])
    wv, bv = params['V'][0].T, r2(params['V'][1])
    wl, bl = r2(params['label'][0][:, 0]), r2(params['label'][1])
    p0, p1 = params['pred']
    pg1, pb1 = r2(p0['ln'][0]), r2(p0['ln'][1])
    w11, b11 = p0['l1'][0].T, r2(p0['l1'][1])
    w21, b21 = p0['l2'][0].T, r2(p0['l2'][1])
    pg2, pb2 = r2(p1['ln'][0]), r2(p1['ln'][1])
    w12, b12 = p1['l1'][0].T, r2(p1['l1'][1])
    w22, b22 = p1['l2'][0].T, r2(p1['l2'][1])
    hg, hb = r2(params['head_ln'][0]), r2(params['head_ln'][1])
    wh, bh = params['head'][0].T, r2(params['head'][1])

    cx = jnp.pad(candidate_x_num, ((0, CPAD - NC), (0, 0)))
    cy = jnp.pad(candidate_y, (0, CPAD - NC)).reshape(CPAD, 1)
    nct = CPAD // CT
    nqt = B // QT

    enc_w = [wlin, blin, w1a, b1a, w2a, b2a, g1, be1, w1b, b1b, w2b, b2b,
             gn, bn]
    enc_specs = [_full(w.shape) for w in enc_w]

    cand_xn, cand_k = pl.pallas_call(
        _encode_body,
        grid=(nct,),
        in_specs=[pl.BlockSpec((CT, D_IN), lambda i: (i, 0))] + enc_specs
        + [_full(wk.shape), _full(bk.shape)],
        out_specs=[pl.BlockSpec((CT, D), lambda i: (i, 0))] * 2,
        out_shape=[jax.ShapeDtypeStruct((CPAD, D), jnp.bfloat16),
                   jax.ShapeDtypeStruct((CPAD, D), F32)],
    )(cx, *enc_w, wk, bk)

    xq_specs = [_full((B, D_IN))] + [_full(w.shape) for w in enc_w] \
        + [_full(wq.shape), _full(bq.shape)]
    x_enc, q = pl.pallas_call(
        _encode_q_body,
        grid=(1,),
        in_specs=xq_specs,
        out_specs=[_full((B, D))] * 2,
        out_shape=[jax.ShapeDtypeStruct((B, D), F32)] * 2,
    )(x_num, *enc_w, wq, bq)

    scores = pl.pallas_call(
        _scores_body,
        grid=(nqt, nct),
        in_specs=[pl.BlockSpec((QT, D), lambda i, j: (i, 0)),
                  pl.BlockSpec((CT, D), lambda i, j: (j, 0))],
        out_specs=pl.BlockSpec((QT, CT), lambda i, j: (i, j)),
        out_shape=jax.ShapeDtypeStruct((B, CPAD), F32),
    )(q, cand_k)

    QT2 = 32
    thr, rowmax = pl.pallas_call(
        functools.partial(_thresh_body, k=96),
        grid=(B // QT2,),
        in_specs=[pl.BlockSpec((QT2, CPAD), lambda i: (i, 0))],
        out_specs=[pl.BlockSpec((QT2, 1), lambda i: (i, 0))] * 2,
        out_shape=[jax.ShapeDtypeStruct((B, 1), F32)] * 2,
    )(scores)

    S, wy, den = pl.pallas_call(
        functools.partial(_wsum_body, nj=nct),
        grid=(nqt, nct),
        in_specs=[pl.BlockSpec((QT, CT), lambda i, j: (i, j)),
                  pl.BlockSpec((CT, D), lambda i, j: (j, 0)),
                  pl.BlockSpec((CT, 1), lambda i, j: (j, 0)),
                  pl.BlockSpec((QT, 1), lambda i, j: (i, 0)),
                  pl.BlockSpec((QT, 1), lambda i, j: (i, 0))],
        out_specs=[pl.BlockSpec((QT, D), lambda i, j: (i, 0)),
                   pl.BlockSpec((QT, 1), lambda i, j: (i, 0)),
                   pl.BlockSpec((QT, 1), lambda i, j: (i, 0))],
        out_shape=[jax.ShapeDtypeStruct((B, D), F32),
                   jax.ShapeDtypeStruct((B, 1), F32),
                   jax.ShapeDtypeStruct((B, 1), F32)],
        scratch_shapes=[pltpu.VMEM((QT, D), F32),
                        pltpu.VMEM((QT, 1), F32),
                        pltpu.VMEM((QT, 1), F32)],
    )(scores, cand_xn, cy, thr, rowmax)

    head_w = [wv, bv, wl, bl, pg1, pb1, w11, b11, w21, b21,
              pg2, pb2, w12, b12, w22, b22, hg, hb, wh, bh]
    out = pl.pallas_call(
        _head_body,
        grid=(nqt,),
        in_specs=[pl.BlockSpec((QT, D), lambda i: (i, 0)),
                  pl.BlockSpec((QT, D), lambda i: (i, 0)),
                  pl.BlockSpec((QT, 1), lambda i: (i, 0)),
                  pl.BlockSpec((QT, 1), lambda i: (i, 0))]
        + [_full(w.shape) for w in head_w],
        out_specs=pl.BlockSpec((QT, 1), lambda i: (i, 0)),
        out_shape=jax.ShapeDtypeStruct((B, 1), F32),
    )(x_enc, S, wy, den, *head_w)

    return out


# speedup vs baseline: 1.0474x; 1.0474x over previous
import functools
import jax
import jax.numpy as jnp
from jax.experimental import pallas as pl
from jax.experimental.pallas import tpu as pltpu

F32 = jnp.float32
NEG = -3e38
CPAD = 50176
CT = 1024
QT = 128


def _ln(x, g, b):
    m = jnp.mean(x, axis=-1, keepdims=True)
    v = jnp.mean((x - m) ** 2, axis=-1, keepdims=True)
    return (x - m) * jax.lax.rsqrt(v + 1e-5) * g + b


def _encode_body(x_ref, wlin, blin, w1a, b1a, w2a, b2a, g1, be1, w1b, b1b,
                 w2b, b2b, gn, bn, wo, bo, xn_out, o_out):
    h = jnp.dot(x_ref[...], wlin[...], preferred_element_type=F32) + blin[...]
    z = jnp.maximum(jnp.dot(h, w1a[...], preferred_element_type=F32) + b1a[...], 0.0)
    h = h + jnp.dot(z, w2a[...], preferred_element_type=F32) + b2a[...]
    zn = _ln(h, g1[...], be1[...])
    z = jnp.maximum(jnp.dot(zn, w1b[...], preferred_element_type=F32) + b1b[...], 0.0)
    h = h + jnp.dot(z, w2b[...], preferred_element_type=F32) + b2b[...]
    hn = _ln(h, gn[...], bn[...])
    xn_out[...] = hn.astype(jnp.bfloat16)
    o_out[...] = jnp.dot(hn, wo[...], preferred_element_type=F32) + bo[...]


def _encode_q_body(x_ref, wlin, blin, w1a, b1a, w2a, b2a, g1, be1, w1b, b1b,
                   w2b, b2b, gn, bn, wq, bq, x_out, q_out):
    h = jnp.dot(x_ref[...], wlin[...], preferred_element_type=F32) + blin[...]
    z = jnp.maximum(jnp.dot(h, w1a[...], preferred_element_type=F32) + b1a[...], 0.0)
    h = h + jnp.dot(z, w2a[...], preferred_element_type=F32) + b2a[...]
    zn = _ln(h, g1[...], be1[...])
    z = jnp.maximum(jnp.dot(zn, w1b[...], preferred_element_type=F32) + b1b[...], 0.0)
    h = h + jnp.dot(z, w2b[...], preferred_element_type=F32) + b2b[...]
    hn = _ln(h, gn[...], bn[...])
    x_out[...] = h
    q_out[...] = jnp.dot(hn, wq[...], preferred_element_type=F32) + bq[...]


def _scores_body(q_ref, ck_ref, out_ref):
    j = pl.program_id(1)
    s = jax.lax.dot_general(q_ref[...], ck_ref[...],
                            (((1,), (1,)), ((), ())),
                            preferred_element_type=F32)
    col = j * CT + jax.lax.broadcasted_iota(jnp.int32, s.shape, 1)
    out_ref[...] = jnp.where(col >= 50000, NEG, s)


def _inv_sortable(u):
    b = jnp.where(u >= 0, u, u ^ jnp.int32(0x7FFFFFFF))
    return jax.lax.bitcast_convert_type(b, F32)


def _thresh_body(s_ref, t_out, m_out, k):
    s = s_ref[...]
    rows, n = s.shape
    rowmax = jnp.max(s, axis=1, keepdims=True)

    def sortable(x):
        bx = jax.lax.bitcast_convert_type(x, jnp.int32)
        return jnp.where(bx >= 0, bx, bx ^ jnp.int32(0x7FFFFFFF))

    cmax = jnp.max(s.reshape(rows, n // 128, 128), axis=1)

    def body_sm(it, t):
        cand = t + jax.lax.shift_left(jnp.int32(1), 30 - it)
        cnt = jnp.sum((cmax >= _inv_sortable(cand)).astype(F32), axis=1,
                      keepdims=True)
        return jnp.where(cnt >= k, cand, t)

    def body(it, t):
        cand = t + jax.lax.shift_left(jnp.int32(1), pmax - it)
        cnt = jnp.sum((s >= _inv_sortable(cand)).astype(F32), axis=1,
                      keepdims=True)
        return jnp.where(cnt >= k, cand, t)

    cnt0 = jnp.sum((s >= 0.0).astype(F32), axis=1, keepdims=True)
    t0 = jnp.where(cnt0 >= k, jnp.int32(0), jnp.int32(-2147483647 - 1))
    u_lo = jax.lax.fori_loop(0, 31, body_sm, t0)
    u_hi = sortable(rowmax)
    u_hi = jnp.where(t0 < 0, jnp.minimum(u_hi, -1), u_hi)
    u_hi = jnp.maximum(u_hi, u_lo)
    gap = (u_lo ^ u_hi).astype(F32)
    e = (jax.lax.bitcast_convert_type(jnp.max(gap), jnp.int32) >> 23) - 127
    pmax = jnp.clip(e + 1, 0, 30)
    mask_hi = ~(jax.lax.shift_left(jnp.int32(2), pmax) - 1)
    t = jax.lax.fori_loop(0, pmax + 1, body, u_lo & mask_hi)
    t_out[...] = _inv_sortable(t)
    m_out[...] = rowmax


def _wsum_body(s_ref, xn_ref, y_ref, t_ref, m_ref, s_out, wy_out, d_out,
               s_acc, wy_acc, d_acc, nj):
    j = pl.program_id(1)

    @pl.when(j == 0)
    def _():
        s_acc[...] = jnp.zeros_like(s_acc)
        wy_acc[...] = jnp.zeros_like(wy_acc)
        d_acc[...] = jnp.zeros_like(d_acc)

    s = s_ref[...]
    w = jnp.where(s >= t_ref[...],
                  jnp.exp((s - m_ref[...]) * (1.0 / 16.0)), 0.0)
    s_acc[...] += jnp.dot(w.astype(jnp.bfloat16), xn_ref[...],
                          preferred_element_type=F32)
    wy_acc[...] += jnp.dot(w, y_ref[...], preferred_element_type=F32)
    d_acc[...] += jnp.sum(w, axis=1, keepdims=True)

    @pl.when(j == nj - 1)
    def _():
        s_out[...] = s_acc[...]
        wy_out[...] = wy_acc[...]
        d_out[...] = d_acc[...]


def _head_body(x_ref, s_ref, wy_ref, d_ref, wv, bv, wl, bl,
               g1, be1, w11, b11, w21, b21, g2, be2, w12, b12, w22, b22,
               hg, hb, wh, bh, out_ref):
    dinv = 1.0 / d_ref[...]
    sn = s_ref[...] * dinv
    ctx = (jnp.dot(sn, wv[...], preferred_element_type=F32) + bv[...]
           + (wy_ref[...] * dinv) * wl[...] + bl[...])
    h = x_ref[...] + ctx
    z = _ln(h, g1[...], be1[...])
    z = jnp.maximum(jnp.dot(z, w11[...], preferred_element_type=F32) + b11[...], 0.0)
    h = h + jnp.dot(z, w21[...], preferred_element_type=F32) + b21[...]
    z = _ln(h, g2[...], be2[...])
    z = jnp.maximum(jnp.dot(z, w12[...], preferred_element_type=F32) + b12[...], 0.0)
    h = h + jnp.dot(z, w22[...], preferred_element_type=F32) + b22[...]
    hn = jnp.maximum(_ln(h, hg[...], hb[...]), 0.0)
    out_ref[...] = jnp.dot(hn, wh[...], preferred_element_type=F32) + bh[...]


def _full(shape):
    n = len(shape)
    return pl.BlockSpec(shape, lambda *a: (0,) * n)


def kernel(x_num, candidate_x_num, candidate_y, params, context_size):
    B, D_IN = x_num.shape
    NC = candidate_x_num.shape[0]
    D = params['norm'][0].shape[0]

    r2 = lambda v: v.reshape(1, -1)
    wlin, blin = params['lin'][0].T, r2(params['lin'][1])
    e0, e1 = params['enc']
    w1a, b1a = e0['l1'][0].T, r2(e0['l1'][1])
    w2a, b2a = e0['l2'][0].T, r2(e0['l2'][1])
    g1, be1 = r2(e1['ln'][0]), r2(e1['ln'][1])
    w1b, b1b = e1['l1'][0].T, r2(e1['l1'][1])
    w2b, b2b = e1['l2'][0].T, r2(e1['l2'][1])
    gn, bn = r2(params['norm'][0]), r2(params['norm'][1])
    wk, bk = params['K'][0].T, r2(params['K'][1])
    wq, bq = params['Q'][0].T, r2(params['Q'][1])
    wv, bv = params['V'][0].T, r2(params['V'][1])
    wl, bl = r2(params['label'][0][:, 0]), r2(params['label'][1])
    p0, p1 = params['pred']
    pg1, pb1 = r2(p0['ln'][0]), r2(p0['ln'][1])
    w11, b11 = p0['l1'][0].T, r2(p0['l1'][1])
    w21, b21 = p0['l2'][0].T, r2(p0['l2'][1])
    pg2, pb2 = r2(p1['ln'][0]), r2(p1['ln'][1])
    w12, b12 = p1['l1'][0].T, r2(p1['l1'][1])
    w22, b22 = p1['l2'][0].T, r2(p1['l2'][1])
    hg, hb = r2(params['head_ln'][0]), r2(params['head_ln'][1])
    wh, bh = params['head'][0].T, r2(params['head'][1])

    cx = jnp.pad(candidate_x_num, ((0, CPAD - NC), (0, 0)))
    cy = jnp.pad(candidate_y, (0, CPAD - NC)).reshape(CPAD, 1)
    nct = CPAD // CT
    nqt = B // QT

    enc_w = [wlin, blin, w1a, b1a, w2a, b2a, g1, be1, w1b, b1b, w2b, b2b,
             gn, bn]
    enc_specs = [_full(w.shape) for w in enc_w]

    cand_xn, cand_k = pl.pallas_call(
        _encode_body,
        grid=(nct,),
        in_specs=[pl.BlockSpec((CT, D_IN), lambda i: (i, 0))] + enc_specs
        + [_full(wk.shape), _full(bk.shape)],
        out_specs=[pl.BlockSpec((CT, D), lambda i: (i, 0))] * 2,
        out_shape=[jax.ShapeDtypeStruct((CPAD, D), jnp.bfloat16),
                   jax.ShapeDtypeStruct((CPAD, D), F32)],
    )(cx, *enc_w, wk, bk)

    xq_specs = [_full((B, D_IN))] + [_full(w.shape) for w in enc_w] \
        + [_full(wq.shape), _full(bq.shape)]
    x_enc, q = pl.pallas_call(
        _encode_q_body,
        grid=(1,),
        in_specs=xq_specs,
        out_specs=[_full((B, D))] * 2,
        out_shape=[jax.ShapeDtypeStruct((B, D), F32)] * 2,
    )(x_num, *enc_w, wq, bq)

    scores = pl.pallas_call(
        _scores_body,
        grid=(nqt, nct),
        in_specs=[pl.BlockSpec((QT, D), lambda i, j: (i, 0)),
                  pl.BlockSpec((CT, D), lambda i, j: (j, 0))],
        out_specs=pl.BlockSpec((QT, CT), lambda i, j: (i, j)),
        out_shape=jax.ShapeDtypeStruct((B, CPAD), F32),
    )(q, cand_k)

    QT2 = 32
    thr, rowmax = pl.pallas_call(
        functools.partial(_thresh_body, k=96),
        grid=(B // QT2,),
        in_specs=[pl.BlockSpec((QT2, CPAD), lambda i: (i, 0))],
        out_specs=[pl.BlockSpec((QT2, 1), lambda i: (i, 0))] * 2,
        out_shape=[jax.ShapeDtypeStruct((B, 1), F32)] * 2,
    )(scores)

    S, wy, den = pl.pallas_call(
        functools.partial(_wsum_body, nj=nct),
        grid=(nqt, nct),
        in_specs=[pl.BlockSpec((QT, CT), lambda i, j: (i, j)),
                  pl.BlockSpec((CT, D), lambda i, j: (j, 0)),
                  pl.BlockSpec((CT, 1), lambda i, j: (j, 0)),
                  pl.BlockSpec((QT, 1), lambda i, j: (i, 0)),
                  pl.BlockSpec((QT, 1), lambda i, j: (i, 0))],
        out_specs=[pl.BlockSpec((QT, D), lambda i, j: (i, 0)),
                   pl.BlockSpec((QT, 1), lambda i, j: (i, 0)),
                   pl.BlockSpec((QT, 1), lambda i, j: (i, 0))],
        out_shape=[jax.ShapeDtypeStruct((B, D), F32),
                   jax.ShapeDtypeStruct((B, 1), F32),
                   jax.ShapeDtypeStruct((B, 1), F32)],
        scratch_shapes=[pltpu.VMEM((QT, D), F32),
                        pltpu.VMEM((QT, 1), F32),
                        pltpu.VMEM((QT, 1), F32)],
    )(scores, cand_xn, cy, thr, rowmax)

    head_w = [wv, bv, wl, bl, pg1, pb1, w11, b11, w21, b21,
              pg2, pb2, w12, b12, w22, b22, hg, hb, wh, bh]
    out = pl.pallas_call(
        _head_body,
        grid=(nqt,),
        in_specs=[pl.BlockSpec((QT, D), lambda i: (i, 0)),
                  pl.BlockSpec((QT, D), lambda i: (i, 0)),
                  pl.BlockSpec((QT, 1), lambda i: (i, 0)),
                  pl.BlockSpec((QT, 1), lambda i: (i, 0))]
        + [_full(w.shape) for w in head_w],
        out_specs=pl.BlockSpec((QT, 1), lambda i: (i, 0)),
        out_shape=jax.ShapeDtypeStruct((B, 1), F32),
    )(x_enc, S, wy, den, *head_w)

    return out


# speedup vs baseline: 1.0507x; 1.0031x over previous
import functools
import jax
import jax.numpy as jnp
from jax.experimental import pallas as pl
from jax.experimental.pallas import tpu as pltpu

F32 = jnp.float32
NEG = -3e38
CPAD = 50176
CT = 1024
QT = 128


def _ln(x, g, b):
    m = jnp.mean(x, axis=-1, keepdims=True)
    v = jnp.mean((x - m) ** 2, axis=-1, keepdims=True)
    return (x - m) * jax.lax.rsqrt(v + 1e-5) * g + b


def _encode_body(x_ref, wlin, blin, w1a, b1a, w2a, b2a, g1, be1, w1b, b1b,
                 w2b, b2b, gn, bn, wo, bo, xn_out, o_out):
    h = jnp.dot(x_ref[...], wlin[...], preferred_element_type=F32) + blin[...]
    z = jnp.maximum(jnp.dot(h, w1a[...], preferred_element_type=F32) + b1a[...], 0.0)
    h = h + jnp.dot(z, w2a[...], preferred_element_type=F32) + b2a[...]
    zn = _ln(h, g1[...], be1[...])
    z = jnp.maximum(jnp.dot(zn, w1b[...], preferred_element_type=F32) + b1b[...], 0.0)
    h = h + jnp.dot(z, w2b[...], preferred_element_type=F32) + b2b[...]
    hn = _ln(h, gn[...], bn[...])
    xn_out[...] = hn.astype(jnp.bfloat16)
    o_out[...] = jnp.dot(hn, wo[...], preferred_element_type=F32) + bo[...]


def _encode_q_body(x_ref, wlin, blin, w1a, b1a, w2a, b2a, g1, be1, w1b, b1b,
                   w2b, b2b, gn, bn, wq, bq, x_out, q_out):
    h = jnp.dot(x_ref[...], wlin[...], preferred_element_type=F32) + blin[...]
    z = jnp.maximum(jnp.dot(h, w1a[...], preferred_element_type=F32) + b1a[...], 0.0)
    h = h + jnp.dot(z, w2a[...], preferred_element_type=F32) + b2a[...]
    zn = _ln(h, g1[...], be1[...])
    z = jnp.maximum(jnp.dot(zn, w1b[...], preferred_element_type=F32) + b1b[...], 0.0)
    h = h + jnp.dot(z, w2b[...], preferred_element_type=F32) + b2b[...]
    hn = _ln(h, gn[...], bn[...])
    x_out[...] = h
    q_out[...] = jnp.dot(hn, wq[...], preferred_element_type=F32) + bq[...]


def _scores_body(q_ref, ck_ref, out_ref):
    j = pl.program_id(1)
    s = jax.lax.dot_general(q_ref[...], ck_ref[...],
                            (((1,), (1,)), ((), ())),
                            preferred_element_type=F32)
    col = j * CT + jax.lax.broadcasted_iota(jnp.int32, s.shape, 1)
    out_ref[...] = jnp.where(col >= 50000, NEG, s)


def _inv_sortable(u):
    b = jnp.where(u >= 0, u, u ^ jnp.int32(0x7FFFFFFF))
    return jax.lax.bitcast_convert_type(b, F32)


def _thresh_body(s_ref, t_out, m_out, k):
    s = s_ref[...]
    rows, n = s.shape

    def sortable(x):
        bx = jax.lax.bitcast_convert_type(x, jnp.int32)
        return jnp.where(bx >= 0, bx, bx ^ jnp.int32(0x7FFFFFFF))

    cmax = jnp.max(s.reshape(rows, n // 128, 128), axis=1)
    rowmax = jnp.max(cmax, axis=1, keepdims=True)

    def body_sm(it, t):
        cand = t + jax.lax.shift_left(jnp.int32(1), 30 - it)
        cnt = jnp.sum((cmax >= _inv_sortable(cand)).astype(F32), axis=1,
                      keepdims=True)
        return jnp.where(cnt >= k, cand, t)

    def body(it, t):
        cand = t + jax.lax.shift_left(jnp.int32(1), pmax - it)
        cnt = jnp.sum((s >= _inv_sortable(cand)).astype(F32), axis=1,
                      keepdims=True)
        return jnp.where(cnt >= k, cand, t)

    cnt0 = jnp.sum((s >= 0.0).astype(F32), axis=1, keepdims=True)
    t0 = jnp.where(cnt0 >= k, jnp.int32(0), jnp.int32(-2147483647 - 1))
    u_lo = jax.lax.fori_loop(0, 31, body_sm, t0)
    u_hi = sortable(rowmax)
    u_hi = jnp.where(t0 < 0, jnp.minimum(u_hi, -1), u_hi)
    u_hi = jnp.maximum(u_hi, u_lo)
    gap = (u_lo ^ u_hi).astype(F32)
    e = (jax.lax.bitcast_convert_type(jnp.max(gap), jnp.int32) >> 23) - 127
    pmax = jnp.clip(e + 1, 0, 30)
    mask_hi = ~(jax.lax.shift_left(jnp.int32(2), pmax) - 1)
    t = jax.lax.fori_loop(0, pmax + 1, body, u_lo & mask_hi)
    t_out[...] = _inv_sortable(t)
    m_out[...] = rowmax


def _wsum_body(s_ref, xn_ref, y_ref, t_ref, m_ref, s_out, wy_out, d_out,
               s_acc, wy_acc, d_acc, nj):
    j = pl.program_id(1)

    @pl.when(j == 0)
    def _():
        s_acc[...] = jnp.zeros_like(s_acc)
        wy_acc[...] = jnp.zeros_like(wy_acc)
        d_acc[...] = jnp.zeros_like(d_acc)

    s = s_ref[...]
    w = jnp.where(s >= t_ref[...],
                  jnp.exp((s - m_ref[...]) * (1.0 / 16.0)), 0.0)
    s_acc[...] += jnp.dot(w.astype(jnp.bfloat16), xn_ref[...],
                          preferred_element_type=F32)
    wy_acc[...] += jnp.dot(w, y_ref[...], preferred_element_type=F32)
    d_acc[...] += jnp.sum(w, axis=1, keepdims=True)

    @pl.when(j == nj - 1)
    def _():
        s_out[...] = s_acc[...]
        wy_out[...] = wy_acc[...]
        d_out[...] = d_acc[...]


def _head_body(x_ref, s_ref, wy_ref, d_ref, wv, bv, wl, bl,
               g1, be1, w11, b11, w21, b21, g2, be2, w12, b12, w22, b22,
               hg, hb, wh, bh, out_ref):
    dinv = 1.0 / d_ref[...]
    sn = s_ref[...] * dinv
    ctx = (jnp.dot(sn, wv[...], preferred_element_type=F32) + bv[...]
           + (wy_ref[...] * dinv) * wl[...] + bl[...])
    h = x_ref[...] + ctx
    z = _ln(h, g1[...], be1[...])
    z = jnp.maximum(jnp.dot(z, w11[...], preferred_element_type=F32) + b11[...], 0.0)
    h = h + jnp.dot(z, w21[...], preferred_element_type=F32) + b21[...]
    z = _ln(h, g2[...], be2[...])
    z = jnp.maximum(jnp.dot(z, w12[...], preferred_element_type=F32) + b12[...], 0.0)
    h = h + jnp.dot(z, w22[...], preferred_element_type=F32) + b22[...]
    hn = jnp.maximum(_ln(h, hg[...], hb[...]), 0.0)
    out_ref[...] = jnp.dot(hn, wh[...], preferred_element_type=F32) + bh[...]


def _full(shape):
    n = len(shape)
    return pl.BlockSpec(shape, lambda *a: (0,) * n)


def kernel(x_num, candidate_x_num, candidate_y, params, context_size):
    B, D_IN = x_num.shape
    NC = candidate_x_num.shape[0]
    D = params['norm'][0].shape[0]

    r2 = lambda v: v.reshape(1, -1)
    wlin, blin = params['lin'][0].T, r2(params['lin'][1])
    e0, e1 = params['enc']
    w1a, b1a = e0['l1'][0].T, r2(e0['l1'][1])
    w2a, b2a = e0['l2'][0].T, r2(e0['l2'][1])
    g1, be1 = r2(e1['ln'][0]), r2(e1['ln'][1])
    w1b, b1b = e1['l1'][0].T, r2(e1['l1'][1])
    w2b, b2b = e1['l2'][0].T, r2(e1['l2'][1])
    gn, bn = r2(params['norm'][0]), r2(params['norm'][1])
    wk, bk = params['K'][0].T, r2(params['K'][1])
    wq, bq = params['Q'][0].T, r2(params['Q'][1])
    wv, bv = params['V'][0].T, r2(params['V'][1])
    wl, bl = r2(params['label'][0][:, 0]), r2(params['label'][1])
    p0, p1 = params['pred']
    pg1, pb1 = r2(p0['ln'][0]), r2(p0['ln'][1])
    w11, b11 = p0['l1'][0].T, r2(p0['l1'][1])
    w21, b21 = p0['l2'][0].T, r2(p0['l2'][1])
    pg2, pb2 = r2(p1['ln'][0]), r2(p1['ln'][1])
    w12, b12 = p1['l1'][0].T, r2(p1['l1'][1])
    w22, b22 = p1['l2'][0].T, r2(p1['l2'][1])
    hg, hb = r2(params['head_ln'][0]), r2(params['head_ln'][1])
    wh, bh = params['head'][0].T, r2(params['head'][1])

    cx = jnp.pad(candidate_x_num, ((0, CPAD - NC), (0, 0)))
    cy = jnp.pad(candidate_y, (0, CPAD - NC)).reshape(CPAD, 1)
    nct = CPAD // CT
    nqt = B // QT

    enc_w = [wlin, blin, w1a, b1a, w2a, b2a, g1, be1, w1b, b1b, w2b, b2b,
             gn, bn]
    enc_specs = [_full(w.shape) for w in enc_w]

    cand_xn, cand_k = pl.pallas_call(
        _encode_body,
        grid=(nct,),
        in_specs=[pl.BlockSpec((CT, D_IN), lambda i: (i, 0))] + enc_specs
        + [_full(wk.shape), _full(bk.shape)],
        out_specs=[pl.BlockSpec((CT, D), lambda i: (i, 0))] * 2,
        out_shape=[jax.ShapeDtypeStruct((CPAD, D), jnp.bfloat16),
                   jax.ShapeDtypeStruct((CPAD, D), F32)],
    )(cx, *enc_w, wk, bk)

    xq_specs = [_full((B, D_IN))] + [_full(w.shape) for w in enc_w] \
        + [_full(wq.shape), _full(bq.shape)]
    x_enc, q = pl.pallas_call(
        _encode_q_body,
        grid=(1,),
        in_specs=xq_specs,
        out_specs=[_full((B, D))] * 2,
        out_shape=[jax.ShapeDtypeStruct((B, D), F32)] * 2,
    )(x_num, *enc_w, wq, bq)

    scores = pl.pallas_call(
        _scores_body,
        grid=(nqt, nct),
        in_specs=[pl.BlockSpec((QT, D), lambda i, j: (i, 0)),
                  pl.BlockSpec((CT, D), lambda i, j: (j, 0))],
        out_specs=pl.BlockSpec((QT, CT), lambda i, j: (i, j)),
        out_shape=jax.ShapeDtypeStruct((B, CPAD), F32),
    )(q, cand_k)

    QT2 = 32
    thr, rowmax = pl.pallas_call(
        functools.partial(_thresh_body, k=96),
        grid=(B // QT2,),
        in_specs=[pl.BlockSpec((QT2, CPAD), lambda i: (i, 0))],
        out_specs=[pl.BlockSpec((QT2, 1), lambda i: (i, 0))] * 2,
        out_shape=[jax.ShapeDtypeStruct((B, 1), F32)] * 2,
    )(scores)

    S, wy, den = pl.pallas_call(
        functools.partial(_wsum_body, nj=nct),
        grid=(nqt, nct),
        in_specs=[pl.BlockSpec((QT, CT), lambda i, j: (i, j)),
                  pl.BlockSpec((CT, D), lambda i, j: (j, 0)),
                  pl.BlockSpec((CT, 1), lambda i, j: (j, 0)),
                  pl.BlockSpec((QT, 1), lambda i, j: (i, 0)),
                  pl.BlockSpec((QT, 1), lambda i, j: (i, 0))],
        out_specs=[pl.BlockSpec((QT, D), lambda i, j: (i, 0)),
                   pl.BlockSpec((QT, 1), lambda i, j: (i, 0)),
                   pl.BlockSpec((QT, 1), lambda i, j: (i, 0))],
        out_shape=[jax.ShapeDtypeStruct((B, D), F32),
                   jax.ShapeDtypeStruct((B, 1), F32),
                   jax.ShapeDtypeStruct((B, 1), F32)],
        scratch_shapes=[pltpu.VMEM((QT, D), F32),
                        pltpu.VMEM((QT, 1), F32),
                        pltpu.VMEM((QT, 1), F32)],
    )(scores, cand_xn, cy, thr, rowmax)

    head_w = [wv, bv, wl, bl, pg1, pb1, w11, b11, w21, b21,
              pg2, pb2, w12, b12, w22, b22, hg, hb, wh, bh]
    out = pl.pallas_call(
        _head_body,
        grid=(nqt,),
        in_specs=[pl.BlockSpec((QT, D), lambda i: (i, 0)),
                  pl.BlockSpec((QT, D), lambda i: (i, 0)),
                  pl.BlockSpec((QT, 1), lambda i: (i, 0)),
                  pl.BlockSpec((QT, 1), lambda i: (i, 0))]
        + [_full(w.shape) for w in head_w],
        out_specs=pl.BlockSpec((QT, 1), lambda i: (i, 0)),
        out_shape=jax.ShapeDtypeStruct((B, 1), F32),
    )(x_enc, S, wy, den, *head_w)

    return out


# speedup vs baseline: 1.1784x; 1.1216x over previous
import functools
import jax
import jax.numpy as jnp
from jax.experimental import pallas as pl
from jax.experimental.pallas import tpu as pltpu

F32 = jnp.float32
NEG = -3e38
CPAD = 50176
CT = 1024
QT = 128


def _ln(x, g, b):
    m = jnp.mean(x, axis=-1, keepdims=True)
    v = jnp.mean((x - m) ** 2, axis=-1, keepdims=True)
    return (x - m) * jax.lax.rsqrt(v + 1e-5) * g + b


def _encode_body(x_ref, wlin, blin, w1a, b1a, w2a, b2a, g1, be1, w1b, b1b,
                 w2b, b2b, gn, bn, wo, bo, xn_out, o_out):
    h = jnp.dot(x_ref[...], wlin[...], preferred_element_type=F32) + blin[...]
    z = jnp.maximum(jnp.dot(h, w1a[...], preferred_element_type=F32) + b1a[...], 0.0)
    h = h + jnp.dot(z, w2a[...], preferred_element_type=F32) + b2a[...]
    zn = _ln(h, g1[...], be1[...])
    z = jnp.maximum(jnp.dot(zn, w1b[...], preferred_element_type=F32) + b1b[...], 0.0)
    h = h + jnp.dot(z, w2b[...], preferred_element_type=F32) + b2b[...]
    hn = _ln(h, gn[...], bn[...])
    xn_out[...] = hn.astype(jnp.bfloat16)
    o_out[...] = jnp.dot(hn, wo[...], preferred_element_type=F32) + bo[...]


def _encode_q_body(x_ref, wlin, blin, w1a, b1a, w2a, b2a, g1, be1, w1b, b1b,
                   w2b, b2b, gn, bn, wq, bq, x_out, q_out):
    h = jnp.dot(x_ref[...], wlin[...], preferred_element_type=F32) + blin[...]
    z = jnp.maximum(jnp.dot(h, w1a[...], preferred_element_type=F32) + b1a[...], 0.0)
    h = h + jnp.dot(z, w2a[...], preferred_element_type=F32) + b2a[...]
    zn = _ln(h, g1[...], be1[...])
    z = jnp.maximum(jnp.dot(zn, w1b[...], preferred_element_type=F32) + b1b[...], 0.0)
    h = h + jnp.dot(z, w2b[...], preferred_element_type=F32) + b2b[...]
    hn = _ln(h, gn[...], bn[...])
    x_out[...] = h
    q_out[...] = jnp.dot(hn, wq[...], preferred_element_type=F32) + bq[...]


def _scores_body(q_ref, ck_ref, out_ref):
    j = pl.program_id(1)
    s = jax.lax.dot_general(q_ref[...], ck_ref[...],
                            (((1,), (1,)), ((), ())),
                            preferred_element_type=F32)
    col = j * CT + jax.lax.broadcasted_iota(jnp.int32, s.shape, 1)
    out_ref[...] = jnp.where(col >= 50000, NEG, s)


def _inv_sortable(u):
    b = jnp.where(u >= 0, u, u ^ jnp.int32(0x7FFFFFFF))
    return jax.lax.bitcast_convert_type(b, F32)


def _thresh_body(s_ref, t_out, m_out, k):
    s = s_ref[...]
    rows, n = s.shape

    def sortable(x):
        bx = jax.lax.bitcast_convert_type(x, jnp.int32)
        return jnp.where(bx >= 0, bx, bx ^ jnp.int32(0x7FFFFFFF))

    cmax = jnp.max(s.reshape(rows, n // 128, 128), axis=1)
    rowmax = jnp.max(cmax, axis=1, keepdims=True)

    def body_sm(it, t):
        cand = t + jax.lax.shift_left(jnp.int32(1), 30 - it)
        cnt = jnp.sum((cmax >= _inv_sortable(cand)).astype(F32), axis=1,
                      keepdims=True)
        return jnp.where(cnt >= k, cand, t)

    def body(it, t):
        cand = t + jax.lax.shift_left(jnp.int32(1), pmax - it)
        cnt = jnp.sum((s >= _inv_sortable(cand)).astype(F32), axis=1,
                      keepdims=True)
        return jnp.where(cnt >= k, cand, t)

    cnt0 = jnp.sum((s >= 0.0).astype(F32), axis=1, keepdims=True)
    t0 = jnp.where(cnt0 >= k, jnp.int32(0), jnp.int32(-2147483647 - 1))
    u_lo = jax.lax.fori_loop(0, 31, body_sm, t0)
    u_hi = sortable(rowmax)
    u_hi = jnp.where(t0 < 0, jnp.minimum(u_hi, -1), u_hi)
    u_hi = jnp.maximum(u_hi, u_lo)
    gap = (u_lo ^ u_hi).astype(F32)
    e = (jax.lax.bitcast_convert_type(jnp.max(gap), jnp.int32) >> 23) - 127
    pmax = jnp.clip(e + 1, 0, 30)
    mask_hi = ~(jax.lax.shift_left(jnp.int32(2), pmax) - 1)
    t = jax.lax.fori_loop(0, pmax + 1, body, u_lo & mask_hi)
    t_out[...] = _inv_sortable(t)
    m_out[...] = rowmax


def _wsum_body(s_ref, xn_ref, y_ref, t_ref, m_ref, s_out, wy_out, d_out,
               s_acc, wy_acc, d_acc, nj):
    j = pl.program_id(1)

    @pl.when(j == 0)
    def _():
        s_acc[...] = jnp.zeros_like(s_acc)
        wy_acc[...] = jnp.zeros_like(wy_acc)
        d_acc[...] = jnp.zeros_like(d_acc)

    s = s_ref[...]
    w = jnp.where(s >= t_ref[...],
                  jnp.exp((s - m_ref[...]) * (1.0 / 16.0)), 0.0)
    s_acc[...] += jnp.dot(w.astype(jnp.bfloat16), xn_ref[...],
                          preferred_element_type=F32)
    wy_acc[...] += jnp.dot(w, y_ref[...], preferred_element_type=F32)
    d_acc[...] += jnp.sum(w, axis=1, keepdims=True)

    @pl.when(j == nj - 1)
    def _():
        s_out[...] = s_acc[...]
        wy_out[...] = wy_acc[...]
        d_out[...] = d_acc[...]


def _head_body(x_ref, s_ref, wy_ref, d_ref, wv, bv, wl, bl,
               g1, be1, w11, b11, w21, b21, g2, be2, w12, b12, w22, b22,
               hg, hb, wh, bh, out_ref):
    dinv = 1.0 / d_ref[...]
    sn = s_ref[...] * dinv
    ctx = (jnp.dot(sn, wv[...], preferred_element_type=F32) + bv[...]
           + (wy_ref[...] * dinv) * wl[...] + bl[...])
    h = x_ref[...] + ctx
    z = _ln(h, g1[...], be1[...])
    z = jnp.maximum(jnp.dot(z, w11[...], preferred_element_type=F32) + b11[...], 0.0)
    h = h + jnp.dot(z, w21[...], preferred_element_type=F32) + b21[...]
    z = _ln(h, g2[...], be2[...])
    z = jnp.maximum(jnp.dot(z, w12[...], preferred_element_type=F32) + b12[...], 0.0)
    h = h + jnp.dot(z, w22[...], preferred_element_type=F32) + b22[...]
    hn = jnp.maximum(_ln(h, hg[...], hb[...]), 0.0)
    out_ref[...] = jnp.dot(hn, wh[...], preferred_element_type=F32) + bh[...]


def _full(shape):
    n = len(shape)
    return pl.BlockSpec(shape, lambda *a: (0,) * n)


def kernel(x_num, candidate_x_num, candidate_y, params, context_size):
    B, D_IN = x_num.shape
    NC = candidate_x_num.shape[0]
    D = params['norm'][0].shape[0]

    r2 = lambda v: v.reshape(1, -1)
    wlin, blin = params['lin'][0].T, r2(params['lin'][1])
    e0, e1 = params['enc']
    w1a, b1a = e0['l1'][0].T, r2(e0['l1'][1])
    w2a, b2a = e0['l2'][0].T, r2(e0['l2'][1])
    g1, be1 = r2(e1['ln'][0]), r2(e1['ln'][1])
    w1b, b1b = e1['l1'][0].T, r2(e1['l1'][1])
    w2b, b2b = e1['l2'][0].T, r2(e1['l2'][1])
    gn, bn = r2(params['norm'][0]), r2(params['norm'][1])
    wk, bk = params['K'][0].T, r2(params['K'][1])
    wq, bq = params['Q'][0].T, r2(params['Q'][1])
    wv, bv = params['V'][0].T, r2(params['V'][1])
    wl, bl = r2(params['label'][0][:, 0]), r2(params['label'][1])
    p0, p1 = params['pred']
    pg1, pb1 = r2(p0['ln'][0]), r2(p0['ln'][1])
    w11, b11 = p0['l1'][0].T, r2(p0['l1'][1])
    w21, b21 = p0['l2'][0].T, r2(p0['l2'][1])
    pg2, pb2 = r2(p1['ln'][0]), r2(p1['ln'][1])
    w12, b12 = p1['l1'][0].T, r2(p1['l1'][1])
    w22, b22 = p1['l2'][0].T, r2(p1['l2'][1])
    hg, hb = r2(params['head_ln'][0]), r2(params['head_ln'][1])
    wh, bh = params['head'][0].T, r2(params['head'][1])

    cx = jnp.pad(candidate_x_num, ((0, CPAD - NC), (0, 0)))
    cy = jnp.pad(candidate_y, (0, CPAD - NC)).reshape(CPAD, 1)
    nct = CPAD // CT
    nqt = B // QT

    enc_w = [wlin, blin, w1a, b1a, w2a, b2a, g1, be1, w1b, b1b, w2b, b2b,
             gn, bn]
    enc_specs = [_full(w.shape) for w in enc_w]

    cand_xn, cand_k = pl.pallas_call(
        _encode_body,
        grid=(nct,),
        in_specs=[pl.BlockSpec((CT, D_IN), lambda i: (i, 0))] + enc_specs
        + [_full(wk.shape), _full(bk.shape)],
        out_specs=[pl.BlockSpec((CT, D), lambda i: (i, 0))] * 2,
        out_shape=[jax.ShapeDtypeStruct((CPAD, D), jnp.bfloat16),
                   jax.ShapeDtypeStruct((CPAD, D), F32)],
    )(cx, *enc_w, wk, bk)

    xq_specs = [_full((B, D_IN))] + [_full(w.shape) for w in enc_w] \
        + [_full(wq.shape), _full(bq.shape)]
    x_enc, q = pl.pallas_call(
        _encode_q_body,
        grid=(1,),
        in_specs=xq_specs,
        out_specs=[_full((B, D))] * 2,
        out_shape=[jax.ShapeDtypeStruct((B, D), F32)] * 2,
    )(x_num, *enc_w, wq, bq)

    scores = pl.pallas_call(
        _scores_body,
        grid=(nqt, nct),
        in_specs=[pl.BlockSpec((QT, D), lambda i, j: (i, 0)),
                  pl.BlockSpec((CT, D), lambda i, j: (j, 0))],
        out_specs=pl.BlockSpec((QT, CT), lambda i, j: (i, j)),
        out_shape=jax.ShapeDtypeStruct((B, CPAD), F32),
    )(q, cand_k)

    QT2 = 64
    thr, rowmax = pl.pallas_call(
        functools.partial(_thresh_body, k=96),
        grid=(B // QT2,),
        in_specs=[pl.BlockSpec((QT2, CPAD), lambda i: (i, 0))],
        out_specs=[pl.BlockSpec((QT2, 1), lambda i: (i, 0))] * 2,
        out_shape=[jax.ShapeDtypeStruct((B, 1), F32)] * 2,
    )(scores)

    S, wy, den = pl.pallas_call(
        functools.partial(_wsum_body, nj=nct),
        grid=(nqt, nct),
        in_specs=[pl.BlockSpec((QT, CT), lambda i, j: (i, j)),
                  pl.BlockSpec((CT, D), lambda i, j: (j, 0)),
                  pl.BlockSpec((CT, 1), lambda i, j: (j, 0)),
                  pl.BlockSpec((QT, 1), lambda i, j: (i, 0)),
                  pl.BlockSpec((QT, 1), lambda i, j: (i, 0))],
        out_specs=[pl.BlockSpec((QT, D), lambda i, j: (i, 0)),
                   pl.BlockSpec((QT, 1), lambda i, j: (i, 0)),
                   pl.BlockSpec((QT, 1), lambda i, j: (i, 0))],
        out_shape=[jax.ShapeDtypeStruct((B, D), F32),
                   jax.ShapeDtypeStruct((B, 1), F32),
                   jax.ShapeDtypeStruct((B, 1), F32)],
        scratch_shapes=[pltpu.VMEM((QT, D), F32),
                        pltpu.VMEM((QT, 1), F32),
                        pltpu.VMEM((QT, 1), F32)],
    )(scores, cand_xn, cy, thr, rowmax)

    head_w = [wv, bv, wl, bl, pg1, pb1, w11, b11, w21, b21,
              pg2, pb2, w12, b12, w22, b22, hg, hb, wh, bh]
    out = pl.pallas_call(
        _head_body,
        grid=(nqt,),
        in_specs=[pl.BlockSpec((QT, D), lambda i: (i, 0)),
                  pl.BlockSpec((QT, D), lambda i: (i, 0)),
                  pl.BlockSpec((QT, 1), lambda i: (i, 0)),
                  pl.BlockSpec((QT, 1), lambda i: (i, 0))]
        + [_full(w.shape) for w in head_w],
        out_specs=pl.BlockSpec((QT, 1), lambda i: (i, 0)),
        out_shape=jax.ShapeDtypeStruct((B, 1), F32),
    )(x_enc, S, wy, den, *head_w)

    return out
